# Initial kernel scaffold; baseline (speedup 1.0000x reference)
import jax
import jax.numpy as jnp
from jax.experimental import pallas as pl


def kernel(verts, edges, W0, b0, W1, b1):
    raise NotImplementedError("write your pallas kernel here")


# speedup vs baseline: 3.3721x; 3.3721x over previous
import functools

import jax
import jax.numpy as jnp
from jax import lax
from jax.experimental import pallas as pl
from jax.experimental.pallas import tpu as pltpu
from jax.experimental.pallas import tpu_sc as plsc

V = 10000
E = 320000
D = 128
VPAD = 10112
NSUB = 16
NCORE = 2
CHUNK = 128
KB = 16
NBLK = 10
PER_SUB = CHUNK * KB * NBLK
EPAD = PER_SUB * NSUB
RPS = VPAD // NSUB


def _matmuls(verts_pad, Wst, bst):
    def body(v_ref, w_ref, b_ref, o_ref):
        o_ref[0] = (
            jnp.dot(v_ref[...], w_ref[0], preferred_element_type=jnp.float32)
            + b_ref[0]
        )

    BR = 2528
    return pl.pallas_call(
        body,
        grid=(3, VPAD // BR),
        in_specs=[
            pl.BlockSpec((BR, D), lambda j, i: (i, 0)),
            pl.BlockSpec((1, D, D), lambda j, i: (j, 0, 0)),
            pl.BlockSpec((1, 1, D), lambda j, i: (j, 0, 0)),
        ],
        out_specs=pl.BlockSpec((1, BR, D), lambda j, i: (j, i, 0)),
        out_shape=jax.ShapeDtypeStruct((3, VPAD, D), jnp.float32),
    )(verts_pad, Wst, bst)


def _sc_body(init_hbm, table_hbm, ia_hbm, ib_hbm, out_hbm,
             acc, ia_v, ib_v, rows_v, sem):
    c = lax.axis_index("c")
    s = lax.axis_index("s")
    pltpu.sync_copy(init_hbm.at[c, pl.ds(s * RPS, RPS)],
                    acc.at[pl.ds(s * RPS, RPS)])
    plsc.subcore_barrier()

    def blk(kb, carry):
        pltpu.sync_copy(ia_hbm.at[c, s, pl.ds(kb * KB, KB)], ia_v)
        pltpu.sync_copy(ib_hbm.at[c, s, pl.ds(kb * KB, KB)], ib_v)
        for j in range(KB):
            pltpu.async_copy(table_hbm.at[ib_v.at[j]], rows_v, sem).wait()
            pltpu.sync_copy(rows_v, acc.at[ia_v.at[j]], add=True)
        return carry

    lax.fori_loop(0, NBLK, blk, 0)
    plsc.subcore_barrier()
    pltpu.sync_copy(acc.at[pl.ds(s * RPS, RPS)],
                    out_hbm.at[c, pl.ds(s * RPS, RPS)])


def _sc_scatter(init, table, ia3, ib3):
    mesh = plsc.VectorSubcoreMesh(core_axis_name="c", subcore_axis_name="s")
    f = pl.kernel(
        _sc_body,
        out_type=jax.ShapeDtypeStruct((NCORE, VPAD, D), jnp.float32),
        mesh=mesh,
        scratch_types=[
            pltpu.VMEM_SHARED((VPAD, D), jnp.float32),
            pltpu.VMEM((KB, CHUNK), jnp.int32),
            pltpu.VMEM((KB, CHUNK), jnp.int32),
            pltpu.VMEM((CHUNK, D), jnp.float32),
            pltpu.SemaphoreType.DMA,
        ],
    )
    return f(init, table, ia3, ib3)


def _add(a, b):
    def body(a_ref, b_ref, o_ref):
        o_ref[...] = a_ref[...] + b_ref[...]

    BR = 2000
    return pl.pallas_call(
        body,
        grid=(V // BR,),
        in_specs=[
            pl.BlockSpec((BR, D), lambda i: (i, 0)),
            pl.BlockSpec((BR, D), lambda i: (i, 0)),
        ],
        out_specs=pl.BlockSpec((BR, D), lambda i: (i, 0)),
        out_shape=jax.ShapeDtypeStruct((V, D), jnp.float32),
    )(a, b)


def kernel(verts, edges, W0, b0, W1, b1):
    verts_pad = jnp.zeros((VPAD, D), jnp.float32).at[:V].set(verts)
    Wst = jnp.stack([W0, jnp.zeros_like(W0), W1])
    bst = jnp.stack([b0, jnp.zeros_like(b0), b1]).reshape(3, 1, D)
    O = _matmuls(verts_pad, Wst, bst)
    init = O[:2]
    table = O[2]

    e = edges.astype(jnp.int32)
    pad = jnp.full((EPAD - E,), V, jnp.int32)
    ia3 = jnp.stack([
        jnp.concatenate([e[:, 0], pad]),
        jnp.concatenate([e[:, 1], pad]),
    ]).reshape(NCORE, NSUB, KB * NBLK, CHUNK)
    ib3 = jnp.stack([
        jnp.concatenate([e[:, 1], pad]),
        jnp.concatenate([e[:, 0], pad]),
    ]).reshape(NCORE, NSUB, KB * NBLK, CHUNK)

    partials = _sc_scatter(init, table, ia3, ib3)
    return _add(partials[0, :V], partials[1, :V])


# speedup vs baseline: 3.6682x; 1.0878x over previous
import functools

import jax
import jax.numpy as jnp
from jax import lax
from jax.experimental import pallas as pl
from jax.experimental.pallas import tpu as pltpu
from jax.experimental.pallas import tpu_sc as plsc

V = 10000
E = 320000
D = 128
VPAD = 10112
NSUB = 16
NCORE = 2
CHUNK = 128
KB = 16
NBLK = 10
PER_SUB = CHUNK * KB * NBLK
EPAD = PER_SUB * NSUB
RPS = VPAD // NSUB


def _matmuls(verts_pad, Wst, bst):
    def body(v_ref, w_ref, b_ref, o_ref):
        o_ref[0] = (
            jnp.dot(v_ref[...], w_ref[0], preferred_element_type=jnp.float32)
            + b_ref[0]
        )

    BR = 2528
    return pl.pallas_call(
        body,
        grid=(3, VPAD // BR),
        in_specs=[
            pl.BlockSpec((BR, D), lambda j, i: (i, 0)),
            pl.BlockSpec((1, D, D), lambda j, i: (j, 0, 0)),
            pl.BlockSpec((1, 1, D), lambda j, i: (j, 0, 0)),
        ],
        out_specs=pl.BlockSpec((1, BR, D), lambda j, i: (j, i, 0)),
        out_shape=jax.ShapeDtypeStruct((3, VPAD, D), jnp.float32),
    )(verts_pad, Wst, bst)


def _sc_body(init_hbm, table_hbm, ia_hbm, ib_hbm, out_hbm,
             acc, ia_v, ib_v, rows0, rows1, sem0, sem1):
    c = lax.axis_index("c")
    s = lax.axis_index("s")
    pltpu.sync_copy(init_hbm.at[c, pl.ds(s * RPS, RPS)],
                    acc.at[pl.ds(s * RPS, RPS)])
    plsc.subcore_barrier()

    rows = (rows0, rows1)
    sems = (sem0, sem1)

    def blk(kb, carry):
        pltpu.sync_copy(ia_hbm.at[c, s, pl.ds(kb * KB, KB)], ia_v)
        pltpu.sync_copy(ib_hbm.at[c, s, pl.ds(kb * KB, KB)], ib_v)
        d = pltpu.async_copy(table_hbm.at[ib_v.at[0]], rows0, sem0)
        for j in range(KB):
            if j + 1 < KB:
                d_next = pltpu.async_copy(
                    table_hbm.at[ib_v.at[j + 1]], rows[(j + 1) % 2],
                    sems[(j + 1) % 2])
            d.wait()
            pltpu.sync_copy(rows[j % 2], acc.at[ia_v.at[j]], add=True)
            if j + 1 < KB:
                d = d_next
        return carry

    lax.fori_loop(0, NBLK, blk, 0)
    plsc.subcore_barrier()
    pltpu.sync_copy(acc.at[pl.ds(s * RPS, RPS)],
                    out_hbm.at[c, pl.ds(s * RPS, RPS)])


def _sc_scatter(init, table, ia3, ib3):
    mesh = plsc.VectorSubcoreMesh(core_axis_name="c", subcore_axis_name="s")
    f = pl.kernel(
        _sc_body,
        out_type=jax.ShapeDtypeStruct((NCORE, VPAD, D), jnp.float32),
        mesh=mesh,
        scratch_types=[
            pltpu.VMEM_SHARED((VPAD, D), jnp.float32),
            pltpu.VMEM((KB, CHUNK), jnp.int32),
            pltpu.VMEM((KB, CHUNK), jnp.int32),
            pltpu.VMEM((CHUNK, D), jnp.float32),
            pltpu.VMEM((CHUNK, D), jnp.float32),
            pltpu.SemaphoreType.DMA,
            pltpu.SemaphoreType.DMA,
        ],
    )
    return f(init, table, ia3, ib3)


def _add(a, b):
    def body(a_ref, b_ref, o_ref):
        o_ref[...] = a_ref[...] + b_ref[...]

    BR = 2000
    return pl.pallas_call(
        body,
        grid=(V // BR,),
        in_specs=[
            pl.BlockSpec((BR, D), lambda i: (i, 0)),
            pl.BlockSpec((BR, D), lambda i: (i, 0)),
        ],
        out_specs=pl.BlockSpec((BR, D), lambda i: (i, 0)),
        out_shape=jax.ShapeDtypeStruct((V, D), jnp.float32),
    )(a, b)


def kernel(verts, edges, W0, b0, W1, b1):
    verts_pad = jnp.zeros((VPAD, D), jnp.float32).at[:V].set(verts)
    Wst = jnp.stack([W0, jnp.zeros_like(W0), W1])
    bst = jnp.stack([b0, jnp.zeros_like(b0), b1]).reshape(3, 1, D)
    O = _matmuls(verts_pad, Wst, bst)
    init = O[:2]
    table = O[2]

    e = edges.astype(jnp.int32)
    pad = jnp.full((EPAD - E,), V, jnp.int32)
    ia3 = jnp.stack([
        jnp.concatenate([e[:, 0], pad]),
        jnp.concatenate([e[:, 1], pad]),
    ]).reshape(NCORE, NSUB, KB * NBLK, CHUNK)
    ib3 = jnp.stack([
        jnp.concatenate([e[:, 1], pad]),
        jnp.concatenate([e[:, 0], pad]),
    ]).reshape(NCORE, NSUB, KB * NBLK, CHUNK)

    partials = _sc_scatter(init, table, ia3, ib3)
    return _add(partials[0, :V], partials[1, :V])
